# baseline (speedup 1.0000x reference)
import functools

import jax
import jax.numpy as jnp
import numpy as np
from jax import lax
from jax.experimental import pallas as pl
from jax.experimental.pallas import tpu as pltpu
from jax.experimental.pallas import tpu_sc as plsc

_NUM_FIELDS = 26
_EMBED_DIM = 16
_FEATURE_DIM = 416
_BATCH = 4096
_FIELD_SIZE = 38462
_TOTAL_ROWS = _NUM_FIELDS * _FIELD_SIZE
_B_FLAT = _BATCH * _NUM_FIELDS

_NW = 32
_N_PER_W = _B_FLAT // _NW
_CHUNK = 128
_N_CHUNKS = _N_PER_W // _CHUNK
_L = 16

_OFF_FLAT = np.tile(
    np.arange(_NUM_FIELDS, dtype=np.int32) * _FIELD_SIZE, _N_PER_W // _NUM_FIELDS
)


@functools.cache
def _make_sc_gather():
    mesh = plsc.VectorSubcoreMesh(core_axis_name="c", subcore_axis_name="s")

    @functools.partial(
        pl.kernel,
        mesh=mesh,
        compiler_params=pltpu.CompilerParams(use_tc_tiling_on_sc=False),
        out_type=jax.ShapeDtypeStruct((_B_FLAT, _EMBED_DIM), jnp.float32),
        scratch_types=[
            pltpu.VMEM((_N_PER_W,), jnp.int32),
            pltpu.VMEM((_N_PER_W,), jnp.int32),
            pltpu.VMEM((_N_PER_W, _EMBED_DIM), jnp.float32),
            pltpu.SemaphoreType.DMA,
        ],
    )
    def gather_k(ids_hbm, off_hbm, emb_hbm, out_hbm, idx_v, off_v, rows_v, sem):
        wid = lax.axis_index("s") * 2 + lax.axis_index("c")
        base = wid * _N_PER_W
        pltpu.sync_copy(ids_hbm.at[pl.ds(base, _N_PER_W)], idx_v)
        pltpu.sync_copy(off_hbm, off_v)

        def add_off(c, carry):
            sl = pl.ds(c * _L, _L)
            idx_v[sl] = idx_v[sl] + off_v[sl]
            return carry

        lax.fori_loop(0, _N_PER_W // _L, add_off, 0)

        copies = []
        for j in range(_N_CHUNKS):
            copies.append(
                pltpu.async_copy(
                    emb_hbm.at[idx_v.at[pl.ds(j * _CHUNK, _CHUNK)]],
                    rows_v.at[pl.ds(j * _CHUNK, _CHUNK)],
                    sem,
                )
            )
        for cp in copies:
            cp.wait()
        pltpu.sync_copy(rows_v, out_hbm.at[pl.ds(base, _N_PER_W)])

    return gather_k


_TCOLS = 4096


def _transpose_body(e1t_ref, e2t_ref, o1_ref, o2_ref):
    i16 = jnp.eye(_EMBED_DIM, dtype=jnp.float32)
    dn = (((0,), (0,)), ((), ()))
    o1_ref[...] = lax.dot_general(e1t_ref[...], i16, dn,
                                  preferred_element_type=jnp.float32)
    o2_ref[...] = lax.dot_general(e2t_ref[...], i16, dn,
                                  preferred_element_type=jnp.float32)


def _retile_tables(e1t, e2t):
    grid = (pl.cdiv(_TOTAL_ROWS, _TCOLS),)
    in_spec = pl.BlockSpec((_EMBED_DIM, _TCOLS), lambda i: (0, i))
    out_spec = pl.BlockSpec((_TCOLS, _EMBED_DIM), lambda i: (i, 0))
    return pl.pallas_call(
        _transpose_body,
        grid=grid,
        in_specs=[in_spec, in_spec],
        out_specs=[out_spec, out_spec],
        out_shape=[
            jax.ShapeDtypeStruct((_TOTAL_ROWS, _EMBED_DIM), jnp.float32),
            jax.ShapeDtypeStruct((_TOTAL_ROWS, _EMBED_DIM), jnp.float32),
        ],
    )(e1t, e2t)


_BM = 512


def _dense_body(fs_ref, flat_ref, w_ref, b_ref, o_ref):
    g = jax.nn.sigmoid(
        jnp.dot(fs_ref[...], w_ref[...], preferred_element_type=jnp.float32)
        + b_ref[...]
    )
    o_ref[...] = flat_ref[...] * (2.0 * g)


def _dense(fs, flat_emb, W, b):
    grid = (_BATCH // _BM,)
    row_spec = pl.BlockSpec((_BM, _FEATURE_DIM), lambda i: (i, 0))
    full_spec = pl.BlockSpec((_FEATURE_DIM, _FEATURE_DIM), lambda i: (0, 0))
    bias_spec = pl.BlockSpec((1, _FEATURE_DIM), lambda i: (0, 0))
    return pl.pallas_call(
        _dense_body,
        grid=grid,
        in_specs=[row_spec, row_spec, full_spec, bias_spec],
        out_specs=row_spec,
        out_shape=jax.ShapeDtypeStruct((_BATCH, _FEATURE_DIM), jnp.float32),
    )(fs, flat_emb, W, b)


@jax.jit
def kernel(input_ids, flat_emb, emb1, emb2, W1, b1, W2, b2):
    ids_flat = input_ids.reshape(_B_FLAT)
    off = jnp.asarray(_OFF_FLAT)
    gather = _make_sc_gather()
    emb1_rm, emb2_rm = _retile_tables(emb1.T, emb2.T)
    fs1 = gather(ids_flat, off, emb1_rm).reshape(_BATCH, _FEATURE_DIM)
    o1 = _dense(fs1, flat_emb, W1, b1.reshape(1, _FEATURE_DIM))
    fs2 = gather(ids_flat, off, emb2_rm).reshape(_BATCH, _FEATURE_DIM)
    o2 = _dense(fs2, flat_emb, W2, b2.reshape(1, _FEATURE_DIM))
    return (o1, o2)


# speedup vs baseline: 1.3248x; 1.3248x over previous
import functools

import jax
import jax.numpy as jnp
import numpy as np
from jax import lax
from jax.experimental import pallas as pl
from jax.experimental.pallas import tpu as pltpu
from jax.experimental.pallas import tpu_sc as plsc

_NUM_FIELDS = 26
_EMBED_DIM = 16
_FEATURE_DIM = 416
_BATCH = 4096
_FIELD_SIZE = 38462
_TOTAL_ROWS = _NUM_FIELDS * _FIELD_SIZE
_B_FLAT = _BATCH * _NUM_FIELDS

_NW = 32
_N_PER_W = _B_FLAT // _NW
_CHUNK = 128
_N_CHUNKS = _N_PER_W // _CHUNK
_L = 16

_OFF_FLAT = np.tile(
    np.arange(_NUM_FIELDS, dtype=np.int32) * _FIELD_SIZE, _N_PER_W // _NUM_FIELDS
)


@functools.cache
def _make_sc_gather():
    mesh = plsc.VectorSubcoreMesh(core_axis_name="c", subcore_axis_name="s")

    @functools.partial(
        pl.kernel,
        mesh=mesh,
        compiler_params=pltpu.CompilerParams(use_tc_tiling_on_sc=False),
        out_type=[
            jax.ShapeDtypeStruct((_B_FLAT, _EMBED_DIM), jnp.float32),
            jax.ShapeDtypeStruct((_B_FLAT, _EMBED_DIM), jnp.float32),
        ],
        scratch_types=[
            pltpu.VMEM((_N_PER_W,), jnp.int32),
            pltpu.VMEM((_N_PER_W,), jnp.int32),
            pltpu.VMEM((_N_PER_W, _EMBED_DIM), jnp.float32),
            pltpu.SemaphoreType.DMA,
        ],
    )
    def gather_k(ids_hbm, off_hbm, emb1_hbm, emb2_hbm, out1_hbm, out2_hbm,
                 idx_v, off_v, rows_v, sem):
        wid = lax.axis_index("s") * 2 + lax.axis_index("c")
        base = wid * _N_PER_W
        pltpu.sync_copy(ids_hbm.at[pl.ds(base, _N_PER_W)], idx_v)
        pltpu.sync_copy(off_hbm, off_v)

        def add_off(c, carry):
            sl = pl.ds(c * _L, _L)
            idx_v[sl] = idx_v[sl] + off_v[sl]
            return carry

        lax.fori_loop(0, _N_PER_W // _L, add_off, 0)

        for tbl, out in ((emb1_hbm, out1_hbm), (emb2_hbm, out2_hbm)):
            copies = []
            for j in range(_N_CHUNKS):
                copies.append(
                    pltpu.async_copy(
                        tbl.at[idx_v.at[pl.ds(j * _CHUNK, _CHUNK)]],
                        rows_v.at[pl.ds(j * _CHUNK, _CHUNK)],
                        sem,
                    )
                )
            for cp in copies:
                cp.wait()
            pltpu.sync_copy(rows_v, out.at[pl.ds(base, _N_PER_W)])

    return gather_k


_BM = 512


def _dense_body(fs1_ref, fs2_ref, flat_ref, w1_ref, b1_ref, w2_ref, b2_ref,
                o1_ref, o2_ref):
    flat = flat_ref[...]
    g1 = jax.nn.sigmoid(
        jnp.dot(fs1_ref[...], w1_ref[...], preferred_element_type=jnp.float32)
        + b1_ref[...]
    )
    o1_ref[...] = flat * (2.0 * g1)
    g2 = jax.nn.sigmoid(
        jnp.dot(fs2_ref[...], w2_ref[...], preferred_element_type=jnp.float32)
        + b2_ref[...]
    )
    o2_ref[...] = flat * (2.0 * g2)


def _dense(fs1, fs2, flat_emb, W1, b1, W2, b2):
    grid = (_BATCH // _BM,)
    row_spec = pl.BlockSpec((_BM, _FEATURE_DIM), lambda i: (i, 0))
    full_spec = pl.BlockSpec((_FEATURE_DIM, _FEATURE_DIM), lambda i: (0, 0))
    bias_spec = pl.BlockSpec((1, _FEATURE_DIM), lambda i: (0, 0))
    return pl.pallas_call(
        _dense_body,
        grid=grid,
        in_specs=[row_spec, row_spec, row_spec, full_spec, bias_spec,
                  full_spec, bias_spec],
        out_specs=[row_spec, row_spec],
        out_shape=[
            jax.ShapeDtypeStruct((_BATCH, _FEATURE_DIM), jnp.float32),
            jax.ShapeDtypeStruct((_BATCH, _FEATURE_DIM), jnp.float32),
        ],
    )(fs1, fs2, flat_emb, W1, b1, W2, b2)


@jax.jit
def kernel(input_ids, flat_emb, emb1, emb2, W1, b1, W2, b2):
    ids_flat = input_ids.reshape(_B_FLAT)
    fs1_flat, fs2_flat = _make_sc_gather()(ids_flat, _OFF_FLAT, emb1, emb2)
    fs1 = fs1_flat.reshape(_BATCH, _FEATURE_DIM)
    fs2 = fs2_flat.reshape(_BATCH, _FEATURE_DIM)
    o1, o2 = _dense(fs1, fs2, flat_emb, W1,
                    b1.reshape(1, _FEATURE_DIM), W2, b2.reshape(1, _FEATURE_DIM))
    return (o1, o2)
